# baseline (speedup 1.0000x reference)
import functools

import jax
import jax.numpy as jnp
from jax import lax
from jax.experimental import pallas as pl
from jax.experimental.pallas import tpu as pltpu
from jax.experimental.pallas import tpu_sc as plsc

N = 10000
E = 320000
FS = 64
EPS = 1e-5
TE = 4000

_NC = 2
_NS = 16
_CPT = 4
_HALF = E // 2
_CH = 10000
_NCHUNK = _HALF // _CH


def _scatter_max_body(msgT_hbm, dst_hbm, out_hbm, acc_v, ids_v, vals_v):
    c = lax.axis_index("c")
    s = lax.axis_index("s")
    w = s * _NC + c
    g = w // 2
    h = w % 2
    col0 = g * _CPT
    ebase = pl.multiple_of(h * _HALF, 8)
    neg_inf = jnp.full((16,), -jnp.inf, jnp.float32)

    def _ini(j, _):
        acc_v[pl.ds(j * 16, 16)] = neg_inf
        return 0
    lax.fori_loop(0, _CPT * N // 16, _ini, 0)

    def _chunk(cc, _):
        off = pl.multiple_of(ebase + cc * _CH, 8)
        pltpu.sync_copy(dst_hbm.at[pl.ds(off, _CH)], ids_v)
        for k in range(_CPT):
            voff = pl.multiple_of((col0 + k) * E + off, 8)
            pltpu.sync_copy(msgT_hbm.at[pl.ds(voff, _CH)],
                            vals_v.at[pl.ds(k * _CH, _CH)])

        def _step(i, _):
            idx = ids_v[pl.ds(i * 16, 16)]
            for k in range(_CPT):
                fidx = idx + jnp.int32(k * N)
                v = vals_v[pl.ds(k * _CH + i * 16, 16)]
                cur = plsc.load_gather(acc_v, [fidx])
                m = jnp.maximum(cur, v)
                need = v > cur

                def _cond(carry):
                    return jnp.any(carry)

                def _body(carry, fidx=fidx, m=m):
                    plsc.store_scatter(acc_v, [fidx], m, mask=carry)
                    cur2 = plsc.load_gather(acc_v, [fidx])
                    return jnp.logical_and(carry, cur2 < m)

                lax.while_loop(_cond, _body, need)
            return 0
        lax.fori_loop(0, _CH // 16, _step, 0)
        return 0

    lax.fori_loop(0, _NCHUNK, _chunk, 0)

    for k in range(_CPT):
        ooff = pl.multiple_of((h * FS + col0 + k) * N, 8)
        pltpu.sync_copy(acc_v.at[pl.ds(k * N, N)], out_hbm.at[pl.ds(ooff, N)])


def _scatter_max(msgT_flat, dst):
    mesh = plsc.VectorSubcoreMesh(core_axis_name="c", subcore_axis_name="s")
    out = pl.kernel(
        _scatter_max_body,
        out_type=jax.ShapeDtypeStruct((2 * FS * N,), jnp.float32),
        mesh=mesh,
        compiler_params=pltpu.CompilerParams(needs_layout_passes=False),
        scratch_types=[
            pltpu.VMEM((_CPT * N,), jnp.float32),
            pltpu.VMEM((_CH,), jnp.int32),
            pltpu.VMEM((_CPT * _CH,), jnp.float32),
        ],
    )(msgT_flat, dst)
    return out.reshape(2, FS, N)


def _mlp1_body(xi_ref, d_ref, a_ref, c_ref, w_ref, o_ref):
    din = xi_ref.shape[1]
    a = a_ref[0, :]
    c = c_ref[0, :]
    hi = jnp.maximum(xi_ref[...] * a[:din] + c[:din], 0.0)
    hd = jnp.maximum(d_ref[...] * a[din:] + c[din:], 0.0)
    h = jnp.concatenate([hi, hd], axis=1)
    o_ref[...] = jnp.dot(h, w_ref[...], preferred_element_type=jnp.float32,
                         precision=jax.lax.Precision.DEFAULT)


def _mlp2_body(x1_ref, a_ref, c_ref, w_ref, o_ref):
    h = jnp.maximum(x1_ref[...] * a_ref[0, :] + c_ref[0, :], 0.0)
    o_ref[...] = jnp.dot(h, w_ref[...], preferred_element_type=jnp.float32,
                         precision=jax.lax.Precision.DEFAULT)


def _mlp1(xi, d, a, c, w):
    din = xi.shape[1]
    grid = (E // TE,)
    return pl.pallas_call(
        _mlp1_body,
        grid=grid,
        in_specs=[
            pl.BlockSpec((TE, din), lambda i: (i, 0)),
            pl.BlockSpec((TE, din), lambda i: (i, 0)),
            pl.BlockSpec((1, 2 * din), lambda i: (0, 0)),
            pl.BlockSpec((1, 2 * din), lambda i: (0, 0)),
            pl.BlockSpec((2 * din, FS), lambda i: (0, 0)),
        ],
        out_specs=pl.BlockSpec((TE, FS), lambda i: (i, 0)),
        out_shape=jax.ShapeDtypeStruct((E, FS), jnp.float32),
    )(xi, d, a.reshape(1, -1), c.reshape(1, -1), w)


def _mlp2(x1, a, c, w):
    grid = (E // TE,)
    return pl.pallas_call(
        _mlp2_body,
        grid=grid,
        in_specs=[
            pl.BlockSpec((TE, FS), lambda i: (i, 0)),
            pl.BlockSpec((1, FS), lambda i: (0, 0)),
            pl.BlockSpec((1, FS), lambda i: (0, 0)),
            pl.BlockSpec((FS, FS), lambda i: (0, 0)),
        ],
        out_specs=pl.BlockSpec((TE, FS), lambda i: (i, 0)),
        out_shape=jax.ShapeDtypeStruct((E, FS), jnp.float32),
    )(x1, a.reshape(1, -1), c.reshape(1, -1), w)


def _bn_coeffs(m, v, g, b):
    inv = g / jnp.sqrt(v + EPS)
    return inv, b - m * inv


def _edge_conv(x, edge_index, p):
    g1, b1, W1, g2, b2, W2 = p
    src = edge_index[0]
    dst = edge_index[1]
    xi = x[dst]
    d = x[src] - xi
    m1 = jnp.concatenate([jnp.mean(xi, 0), jnp.mean(d, 0)])
    v1 = jnp.concatenate([jnp.var(xi, 0), jnp.var(d, 0)])
    a1, c1 = _bn_coeffs(m1, v1, g1, b1)
    x1 = _mlp1(xi, d, a1, c1, W1)
    a2, c2 = _bn_coeffs(jnp.mean(x1, 0), jnp.var(x1, 0), g2, b2)
    x2 = _mlp2(x1, a2, c2, W2)
    partials = _scatter_max(x2.T.reshape(-1), dst)
    out = jnp.maximum(partials[0], partials[1]).T
    return jnp.where(jnp.isfinite(out), out, 0.0)


def kernel(x, spatial_edge_index, temporal_edge_index, params, fcW, fcb):
    g1s = _edge_conv(x, spatial_edge_index, params[0])
    g1st = _edge_conv(g1s, temporal_edge_index, params[4])
    g2s = _edge_conv(g1st, spatial_edge_index, params[1])
    g2st = _edge_conv(g2s, temporal_edge_index, params[5]) + g1st
    g3s = _edge_conv(g2st, spatial_edge_index, params[2])
    g3st = _edge_conv(g3s, temporal_edge_index, params[6]) + g2st
    g4s = _edge_conv(g3st, spatial_edge_index, params[3])
    g4st = _edge_conv(g4s, temporal_edge_index, params[7]) + g3st
    return jnp.dot(g4st, fcW) + fcb


# speedup vs baseline: 1.4097x; 1.4097x over previous
import functools

import jax
import jax.numpy as jnp
from jax import lax
from jax.experimental import pallas as pl
from jax.experimental.pallas import tpu as pltpu
from jax.experimental.pallas import tpu_sc as plsc

N = 10000
E = 320000
FS = 64
EPS = 1e-5
TE = 4000

_NC = 2
_NS = 16
_CPT = 4
_HALF = E // 2
_CH = 6400
_NCHUNK = _HALF // _CH
_FPC = _CH // 16


def _scatter_max_body(msgT_hbm, dst_hbm, flags_hbm, out_hbm, acc_v, ids_v,
                      vals_v, flags_v):
    c = lax.axis_index("c")
    s = lax.axis_index("s")
    w = s * _NC + c
    g = w // 2
    h = w % 2
    col0 = g * _CPT
    ebase = pl.multiple_of(h * _HALF, 8)
    neg_inf = jnp.full((16,), -jnp.inf, jnp.float32)

    def _ini(j, _):
        acc_v[pl.ds(j * 16, 16)] = neg_inf
        return 0
    lax.fori_loop(0, _CPT * N // 16, _ini, 0)

    def _chunk(cc, _):
        gc = h * _NCHUNK + cc
        off = pl.multiple_of(ebase + cc * _CH, 8)
        pltpu.sync_copy(dst_hbm.at[pl.ds(off, _CH)], ids_v)
        pltpu.sync_copy(flags_hbm.at[pl.ds(pl.multiple_of(gc * _FPC, 8),
                                           _FPC)], flags_v)
        for k in range(_CPT):
            voff = pl.multiple_of((col0 + k) * E + off, 8)
            pltpu.sync_copy(msgT_hbm.at[pl.ds(voff, _CH)],
                            vals_v.at[pl.ds(k * _CH, _CH)])

        def _sstep(j, _):
            anydup = jnp.max(flags_v[pl.ds(j * 16, 16)])

            @pl.when(anydup == 0)
            def _fast():
                for t in range(16):
                    idx = ids_v[pl.ds(j * 256 + t * 16, 16)]
                    for k in range(_CPT):
                        fidx = idx + jnp.int32(k * N)
                        v = vals_v[pl.ds(k * _CH + j * 256 + t * 16, 16)]
                        cur = plsc.load_gather(acc_v, [fidx])
                        plsc.store_scatter(acc_v, [fidx],
                                           jnp.maximum(cur, v))

            @pl.when(anydup != 0)
            def _slow():
                for t in range(16):
                    idx = ids_v[pl.ds(j * 256 + t * 16, 16)]
                    for k in range(_CPT):
                        fidx = idx + jnp.int32(k * N)
                        v = vals_v[pl.ds(k * _CH + j * 256 + t * 16, 16)]
                        cur = plsc.load_gather(acc_v, [fidx])
                        m = jnp.maximum(cur, v)
                        need = v > cur

                        def _cond(carry):
                            return jnp.any(carry)

                        def _body(carry, fidx=fidx, m=m):
                            plsc.store_scatter(acc_v, [fidx], m, mask=carry)
                            cur2 = plsc.load_gather(acc_v, [fidx])
                            return jnp.logical_and(carry, cur2 < m)

                        lax.while_loop(_cond, _body, need)
            return 0
        lax.fori_loop(0, _CH // 256, _sstep, 0)
        return 0

    lax.fori_loop(0, _NCHUNK, _chunk, 0)

    for k in range(_CPT):
        ooff = pl.multiple_of((h * FS + col0 + k) * N, 8)
        pltpu.sync_copy(acc_v.at[pl.ds(k * N, N)], out_hbm.at[pl.ds(ooff, N)])


def _scatter_max(msgT_flat, dst, flags):
    mesh = plsc.VectorSubcoreMesh(core_axis_name="c", subcore_axis_name="s")
    out = pl.kernel(
        _scatter_max_body,
        out_type=jax.ShapeDtypeStruct((2 * FS * N,), jnp.float32),
        mesh=mesh,
        compiler_params=pltpu.CompilerParams(needs_layout_passes=False),
        scratch_types=[
            pltpu.VMEM((_CPT * N,), jnp.float32),
            pltpu.VMEM((_CH,), jnp.int32),
            pltpu.VMEM((_CPT * _CH,), jnp.float32),
            pltpu.VMEM((_FPC,), jnp.int32),
        ],
    )(msgT_flat, dst, flags)
    return out.reshape(2, FS, N)


def _dup_flags(dst):
    srt = jnp.sort(dst.reshape(E // 16, 16), axis=1)
    return jnp.any(srt[:, 1:] == srt[:, :-1], axis=1).astype(jnp.int32)


def _mlp1_body(xi_ref, d_ref, a_ref, c_ref, w_ref, o_ref):
    din = xi_ref.shape[1]
    a = a_ref[0, :]
    c = c_ref[0, :]
    hi = jnp.maximum(xi_ref[...] * a[:din] + c[:din], 0.0)
    hd = jnp.maximum(d_ref[...] * a[din:] + c[din:], 0.0)
    h = jnp.concatenate([hi, hd], axis=1)
    o_ref[...] = jnp.dot(h, w_ref[...], preferred_element_type=jnp.float32,
                         precision=jax.lax.Precision.DEFAULT)


def _mlp2_body(x1_ref, a_ref, c_ref, w_ref, o_ref):
    h = jnp.maximum(x1_ref[...] * a_ref[0, :] + c_ref[0, :], 0.0)
    o_ref[...] = jnp.dot(h, w_ref[...], preferred_element_type=jnp.float32,
                         precision=jax.lax.Precision.DEFAULT)


def _mlp1(xi, d, a, c, w):
    din = xi.shape[1]
    grid = (E // TE,)
    return pl.pallas_call(
        _mlp1_body,
        grid=grid,
        in_specs=[
            pl.BlockSpec((TE, din), lambda i: (i, 0)),
            pl.BlockSpec((TE, din), lambda i: (i, 0)),
            pl.BlockSpec((1, 2 * din), lambda i: (0, 0)),
            pl.BlockSpec((1, 2 * din), lambda i: (0, 0)),
            pl.BlockSpec((2 * din, FS), lambda i: (0, 0)),
        ],
        out_specs=pl.BlockSpec((TE, FS), lambda i: (i, 0)),
        out_shape=jax.ShapeDtypeStruct((E, FS), jnp.float32),
    )(xi, d, a.reshape(1, -1), c.reshape(1, -1), w)


def _mlp2(x1, a, c, w):
    grid = (E // TE,)
    return pl.pallas_call(
        _mlp2_body,
        grid=grid,
        in_specs=[
            pl.BlockSpec((TE, FS), lambda i: (i, 0)),
            pl.BlockSpec((1, FS), lambda i: (0, 0)),
            pl.BlockSpec((1, FS), lambda i: (0, 0)),
            pl.BlockSpec((FS, FS), lambda i: (0, 0)),
        ],
        out_specs=pl.BlockSpec((TE, FS), lambda i: (i, 0)),
        out_shape=jax.ShapeDtypeStruct((E, FS), jnp.float32),
    )(x1, a.reshape(1, -1), c.reshape(1, -1), w)


def _bn_coeffs(m, v, g, b):
    inv = g / jnp.sqrt(v + EPS)
    return inv, b - m * inv


def _edge_conv(x, edge_index, flags, p):
    g1, b1, W1, g2, b2, W2 = p
    src = edge_index[0]
    dst = edge_index[1]
    xi = x[dst]
    d = x[src] - xi
    m1 = jnp.concatenate([jnp.mean(xi, 0), jnp.mean(d, 0)])
    v1 = jnp.concatenate([jnp.var(xi, 0), jnp.var(d, 0)])
    a1, c1 = _bn_coeffs(m1, v1, g1, b1)
    x1 = _mlp1(xi, d, a1, c1, W1)
    a2, c2 = _bn_coeffs(jnp.mean(x1, 0), jnp.var(x1, 0), g2, b2)
    x2 = _mlp2(x1, a2, c2, W2)
    partials = _scatter_max(x2.T.reshape(-1), dst, flags)
    out = jnp.maximum(partials[0], partials[1]).T
    return jnp.where(jnp.isfinite(out), out, 0.0)


def kernel(x, spatial_edge_index, temporal_edge_index, params, fcW, fcb):
    fs = _dup_flags(spatial_edge_index[1])
    ft = _dup_flags(temporal_edge_index[1])
    g1s = _edge_conv(x, spatial_edge_index, fs, params[0])
    g1st = _edge_conv(g1s, temporal_edge_index, ft, params[4])
    g2s = _edge_conv(g1st, spatial_edge_index, fs, params[1])
    g2st = _edge_conv(g2s, temporal_edge_index, ft, params[5]) + g1st
    g3s = _edge_conv(g2st, spatial_edge_index, fs, params[2])
    g3st = _edge_conv(g3s, temporal_edge_index, ft, params[6]) + g2st
    g4s = _edge_conv(g3st, spatial_edge_index, fs, params[3])
    g4st = _edge_conv(g4s, temporal_edge_index, ft, params[7]) + g3st
    return jnp.dot(g4st, fcW) + fcb
